# Initial kernel scaffold; baseline (speedup 1.0000x reference)
import jax
import jax.numpy as jnp
from jax.experimental import pallas as pl


def kernel(features, edge_index, Wa, ba, temp):
    raise NotImplementedError("write your pallas kernel here")


# speedup vs baseline: 8.8999x; 8.8999x over previous
import functools

import jax
import jax.numpy as jnp
from jax import lax
from jax.experimental import pallas as pl
from jax.experimental.pallas import tpu as pltpu
from jax.experimental.pallas import tpu_sc as plsc

_N = 10000
_E = 320000
_D = 128
_DH = _D // 2
_K = 3
_NP = 10240
_EP = 327680
_EPT = _EP // 16
_CH = 128
_NCH = _EPT // _CH
_RPT = _NP // 16


def _sc_body(z_hbm, s1m_hbm, s2b_hbm, mv_hbm, src_hbm, dst_hbm,
             outf_hbm, outd_hbm,
             s1v, s2v, srcv, dstv, mvv, exch, rows, zb, facc, dacc, sem):
  cid = lax.axis_index("c")
  tid = lax.axis_index("s")
  zero16 = jnp.zeros((16,), jnp.float32)

  def _zrow(r, c):
    for i in range(_DH // 16):
      rows[r, pl.ds(i * 16, 16)] = zero16
    return c
  lax.fori_loop(0, _CH, _zrow, 0)

  def _zzb(j, c):
    zb[pl.ds(j * 16, 16)] = zero16
    return c
  lax.fori_loop(0, _RPT // 16, _zzb, 0)

  for b in range(_RPT // _CH):
    pltpu.sync_copy(rows, facc.at[pl.ds(tid * _RPT + b * _CH, _CH)])
  pltpu.sync_copy(zb, dacc.at[pl.ds(tid * _RPT, _RPT)])

  pltpu.sync_copy(s1m_hbm, s1v)
  pltpu.sync_copy(s2b_hbm, s2v)
  pltpu.sync_copy(mv_hbm, mvv)
  pltpu.sync_copy(src_hbm.at[pl.ds(tid * _NCH, _NCH)], srcv)
  pltpu.sync_copy(dst_hbm.at[pl.ds(tid * _NCH, _NCH)], dstv)
  plsc.subcore_barrier()

  mvec = mvv[...]
  nmv = -mvec

  def _chunk(c, carry):
    for i in range(_CH // 16):
      si = srcv[c, pl.ds(i * 16, 16)]
      di = dstv[c, pl.ds(i * 16, 16)]
      t = plsc.load_gather(s1v, [si]) + plsc.load_gather(s2v, [di])
      ex = jnp.exp(jnp.where(t > nmv, t, t * 0.01 - mvec * 0.99))
      exch[pl.ds(i * 16, 16)] = ex
    pltpu.async_copy(z_hbm.at[cid].at[srcv.at[c]], rows, sem).wait()
    def _srow(g, cc):
      wv = exch[pl.ds(g * 16, 16)]
      for jj in range(16):
        r = g * 16 + jj
        w = wv[jj]
        for i in range(_DH // 16):
          rows[r, pl.ds(i * 16, 16)] = rows[r, pl.ds(i * 16, 16)] * w
      return cc
    lax.fori_loop(0, _CH // 16, _srow, 0)
    pltpu.sync_copy(rows, facc.at[dstv.at[c]], add=True)
    pltpu.sync_copy(exch, dacc.at[dstv.at[c]], add=True)
    return carry

  lax.fori_loop(0, _NCH, _chunk, 0)
  plsc.subcore_barrier()

  for b in range(_RPT // _CH):
    off = tid * _RPT + b * _CH
    pltpu.sync_copy(facc.at[pl.ds(off, _CH)], rows)
    pltpu.sync_copy(rows, outf_hbm.at[cid, pl.ds(off, _CH)])
  pltpu.sync_copy(dacc.at[pl.ds(tid * _RPT, _RPT)], zb)
  pltpu.sync_copy(zb, outd_hbm.at[cid, pl.ds(tid * _RPT, _RPT)])


_sc_step = functools.partial(
    pl.kernel,
    out_type=[
        jax.ShapeDtypeStruct((2, _NP, _DH), jnp.float32),
        jax.ShapeDtypeStruct((2, _NP), jnp.float32),
    ],
    mesh=plsc.VectorSubcoreMesh(core_axis_name="c", subcore_axis_name="s"),
    scratch_types=[
        pltpu.VMEM((_NP,), jnp.float32),
        pltpu.VMEM((_NP,), jnp.float32),
        pltpu.VMEM((_NCH, _CH), jnp.int32),
        pltpu.VMEM((_NCH, _CH), jnp.int32),
        pltpu.VMEM((16,), jnp.float32),
        pltpu.VMEM((_CH,), jnp.float32),
        pltpu.VMEM((_CH, _DH), jnp.float32),
        pltpu.VMEM((_RPT,), jnp.float32),
        pltpu.VMEM_SHARED((_NP, _DH), jnp.float32),
        pltpu.VMEM_SHARED((_NP,), jnp.float32),
        pltpu.SemaphoreType.DMA,
    ],
    compiler_params=pltpu.CompilerParams(
        needs_layout_passes=False, use_tc_tiling_on_sc=False),
    name="gat_sc_step",
)(_sc_body)


def _prep_body(z_ref, wa1_ref, wa2_ref, ba_ref, t0_ref,
               s1m_ref, s2b_ref, mv_ref, ft_ref):
  z = z_ref[...]
  s1 = jnp.sum(z * wa1_ref[...], axis=1, keepdims=True)
  s2b = jnp.sum(z * wa2_ref[...], axis=1, keepdims=True) + ba_ref[0, 0]
  m = jnp.maximum(jnp.max(s1) + jnp.max(s2b), 0.0)
  s1m_ref[...] = s1 - m
  s2b_ref[...] = s2b
  mv_ref[...] = jnp.full((1, 16), m, jnp.float32)
  ft_ref[...] = z * t0_ref[0, 0]


_tc_prep = pl.pallas_call(
    _prep_body,
    out_shape=[
        jax.ShapeDtypeStruct((_NP, 1), jnp.float32),
        jax.ShapeDtypeStruct((_NP, 1), jnp.float32),
        jax.ShapeDtypeStruct((1, 16), jnp.float32),
        jax.ShapeDtypeStruct((_NP, _D), jnp.float32),
    ],
    in_specs=[
        pl.BlockSpec(memory_space=pltpu.VMEM),
        pl.BlockSpec(memory_space=pltpu.VMEM),
        pl.BlockSpec(memory_space=pltpu.VMEM),
        pl.BlockSpec(memory_space=pltpu.SMEM),
        pl.BlockSpec(memory_space=pltpu.SMEM),
    ],
    name="gat_tc_prep",
)


def _comb_body(fp_ref, dp_ref, ft_ref, wa1_ref, wa2_ref, ba_ref, tk_ref,
               z3_ref, fto_ref, s1m_ref, s2b_ref, mv_ref):
  fsum = jnp.concatenate([fp_ref[0], fp_ref[1]], axis=1)
  den = dp_ref[0]
  feats = jnp.where(den > 0, fsum / den, 0.0)
  z3_ref[0] = feats[:, :_DH]
  z3_ref[1] = feats[:, _DH:]
  fto_ref[...] = ft_ref[...] + tk_ref[0, 0] * feats
  s1 = jnp.sum(feats * wa1_ref[...], axis=1, keepdims=True)
  s2b = jnp.sum(feats * wa2_ref[...], axis=1, keepdims=True) + ba_ref[0, 0]
  m = jnp.maximum(jnp.max(s1) + jnp.max(s2b), 0.0)
  s1m_ref[...] = s1 - m
  s2b_ref[...] = s2b
  mv_ref[...] = jnp.full((1, 16), m, jnp.float32)


_tc_comb = pl.pallas_call(
    _comb_body,
    out_shape=[
        jax.ShapeDtypeStruct((2, _NP, _DH), jnp.float32),
        jax.ShapeDtypeStruct((_NP, _D), jnp.float32),
        jax.ShapeDtypeStruct((_NP, 1), jnp.float32),
        jax.ShapeDtypeStruct((_NP, 1), jnp.float32),
        jax.ShapeDtypeStruct((1, 16), jnp.float32),
    ],
    in_specs=[
        pl.BlockSpec(memory_space=pltpu.VMEM),
        pl.BlockSpec(memory_space=pltpu.VMEM),
        pl.BlockSpec(memory_space=pltpu.VMEM),
        pl.BlockSpec(memory_space=pltpu.VMEM),
        pl.BlockSpec(memory_space=pltpu.VMEM),
        pl.BlockSpec(memory_space=pltpu.SMEM),
        pl.BlockSpec(memory_space=pltpu.SMEM),
    ],
    compiler_params=pltpu.CompilerParams(vmem_limit_bytes=100 * 1024 * 1024),
    name="gat_tc_comb",
)


def _last_body(fp_ref, dp_ref, ft_ref, tk_ref, fto_ref):
  fsum = jnp.concatenate([fp_ref[0], fp_ref[1]], axis=1)
  den = dp_ref[0]
  feats = jnp.where(den > 0, fsum / den, 0.0)
  fto_ref[...] = ft_ref[...] + tk_ref[0, 0] * feats


_tc_last = pl.pallas_call(
    _last_body,
    out_shape=jax.ShapeDtypeStruct((_NP, _D), jnp.float32),
    in_specs=[
        pl.BlockSpec(memory_space=pltpu.VMEM),
        pl.BlockSpec(memory_space=pltpu.VMEM),
        pl.BlockSpec(memory_space=pltpu.VMEM),
        pl.BlockSpec(memory_space=pltpu.SMEM),
    ],
    name="gat_tc_last",
)


@jax.jit
def _gat(features, edge_index, Wa, ba, temp):
  z0 = jnp.zeros((_NP, _D), jnp.float32).at[:_N].set(features)
  pad = jnp.full((_EP - _E,), _N, jnp.int32)
  src2 = jnp.concatenate([edge_index[0], pad]).reshape(_EP // _CH, _CH)
  dst2 = jnp.concatenate([edge_index[1], pad]).reshape(_EP // _CH, _CH)
  wa1 = Wa[:, :_D]
  wa2 = Wa[:, _D:]
  ba2 = ba.reshape(1, 1)
  s1m, s2b, mv, ft = _tc_prep(z0, wa1, wa2, ba2, temp[0].reshape(1, 1))
  z3 = jnp.stack([z0[:, :_DH], z0[:, _DH:]])
  for k in range(_K):
    fp, dp = _sc_step(z3, s1m.reshape(-1), s2b.reshape(-1), mv.reshape(-1),
                      src2, dst2)
    tk = temp[k + 1].reshape(1, 1)
    if k < _K - 1:
      z3, ft, s1m, s2b, mv = _tc_comb(fp, dp.reshape(2, _NP, 1), ft,
                                      wa1, wa2, ba2, tk)
    else:
      ft = _tc_last(fp, dp.reshape(2, _NP, 1), ft, tk)
  return ft[:_N]


def kernel(features, edge_index, Wa, ba, temp):
  return _gat(features, edge_index, Wa, ba, temp)


# speedup vs baseline: 11.8498x; 1.3315x over previous
import functools

import jax
import jax.numpy as jnp
from jax import lax
from jax.experimental import pallas as pl
from jax.experimental.pallas import tpu as pltpu
from jax.experimental.pallas import tpu_sc as plsc

_N = 10000
_E = 320000
_D = 128
_DH = _D // 2
_K = 3
_NP = 10240
_EP = 327680
_EPT = _EP // 16
_CH = 128
_NCH = _EPT // _CH
_RPT = _NP // 16


def _sc_body(z_hbm, s1m_hbm, s2b_hbm, mv_hbm, src_hbm, dst_hbm,
             outf_hbm, outd_hbm,
             s1v, s2v, srcv, dstv, mvv,
             exch0, exch1, rows0, rows1, zb, facc, dacc,
             gsem0, gsem1, ssem0, ssem1):
  cid = lax.axis_index("c")
  tid = lax.axis_index("s")
  zero16 = jnp.zeros((16,), jnp.float32)

  def _zrow(r, c):
    for i in range(_DH // 16):
      rows0[r, pl.ds(i * 16, 16)] = zero16
    return c
  lax.fori_loop(0, _CH, _zrow, 0)

  def _zzb(j, c):
    zb[pl.ds(j * 16, 16)] = zero16
    return c
  lax.fori_loop(0, _RPT // 16, _zzb, 0)

  for b in range(_RPT // _CH):
    pltpu.sync_copy(rows0, facc.at[pl.ds(tid * _RPT + b * _CH, _CH)])
  pltpu.sync_copy(zb, dacc.at[pl.ds(tid * _RPT, _RPT)])

  pltpu.sync_copy(s1m_hbm, s1v)
  pltpu.sync_copy(s2b_hbm, s2v)
  pltpu.sync_copy(mv_hbm, mvv)
  pltpu.sync_copy(src_hbm.at[pl.ds(tid * _NCH, _NCH)], srcv)
  pltpu.sync_copy(dst_hbm.at[pl.ds(tid * _NCH, _NCH)], dstv)
  plsc.subcore_barrier()

  mvec = mvv[...]
  nmv = -mvec

  def _ex_into(c, exch):
    for i in range(_CH // 16):
      si = srcv[c, pl.ds(i * 16, 16)]
      di = dstv[c, pl.ds(i * 16, 16)]
      t = plsc.load_gather(s1v, [si]) + plsc.load_gather(s2v, [di])
      exch[pl.ds(i * 16, 16)] = jnp.exp(
          jnp.where(t > nmv, t, t * 0.01 - mvec * 0.99))

  def _scale(rows, exch):
    def _srow(g, cc):
      wv = exch[pl.ds(g * 16, 16)]
      for jj in range(16):
        r = g * 16 + jj
        w = wv[jj]
        for i in range(_DH // 16):
          rows[r, pl.ds(i * 16, 16)] = rows[r, pl.ds(i * 16, 16)] * w
      return cc
    lax.fori_loop(0, _CH // 16, _srow, 0)

  def _drain(rows, exch, ssem):
    pltpu.make_async_copy(rows, facc.at[pl.ds(0, _CH)], ssem).wait()
    pltpu.make_async_copy(exch, dacc.at[pl.ds(0, _CH)], ssem).wait()

  def _pair(p, carry):
    a = p * 2
    b = a + 1
    @pl.when(p > 0)
    def _():
      _drain(rows0, exch0, ssem0)
    _ex_into(a, exch0)
    ga = pltpu.async_copy(z_hbm.at[cid].at[srcv.at[a]], rows0, gsem0)
    @pl.when(p > 0)
    def _():
      _drain(rows1, exch1, ssem1)
    _ex_into(b, exch1)
    gb = pltpu.async_copy(z_hbm.at[cid].at[srcv.at[b]], rows1, gsem1)
    ga.wait()
    _scale(rows0, exch0)
    pltpu.async_copy(rows0, facc.at[dstv.at[a]], ssem0, add=True)
    pltpu.async_copy(exch0, dacc.at[dstv.at[a]], ssem0, add=True)
    gb.wait()
    _scale(rows1, exch1)
    pltpu.async_copy(rows1, facc.at[dstv.at[b]], ssem1, add=True)
    pltpu.async_copy(exch1, dacc.at[dstv.at[b]], ssem1, add=True)
    return carry

  lax.fori_loop(0, _NCH // 2, _pair, 0)
  _drain(rows0, exch0, ssem0)
  _drain(rows1, exch1, ssem1)
  plsc.subcore_barrier()

  for b in range(_RPT // _CH):
    off = tid * _RPT + b * _CH
    pltpu.sync_copy(facc.at[pl.ds(off, _CH)], rows0)
    pltpu.sync_copy(rows0, outf_hbm.at[cid, pl.ds(off, _CH)])
  pltpu.sync_copy(dacc.at[pl.ds(tid * _RPT, _RPT)], zb)
  pltpu.sync_copy(zb, outd_hbm.at[cid, pl.ds(tid * _RPT, _RPT)])


_sc_step = functools.partial(
    pl.kernel,
    out_type=[
        jax.ShapeDtypeStruct((2, _NP, _DH), jnp.float32),
        jax.ShapeDtypeStruct((2, _NP), jnp.float32),
    ],
    mesh=plsc.VectorSubcoreMesh(core_axis_name="c", subcore_axis_name="s"),
    scratch_types=[
        pltpu.VMEM((_NP,), jnp.float32),
        pltpu.VMEM((_NP,), jnp.float32),
        pltpu.VMEM((_NCH, _CH), jnp.int32),
        pltpu.VMEM((_NCH, _CH), jnp.int32),
        pltpu.VMEM((16,), jnp.float32),
        pltpu.VMEM((_CH,), jnp.float32),
        pltpu.VMEM((_CH,), jnp.float32),
        pltpu.VMEM((_CH, _DH), jnp.float32),
        pltpu.VMEM((_CH, _DH), jnp.float32),
        pltpu.VMEM((_RPT,), jnp.float32),
        pltpu.VMEM_SHARED((_NP, _DH), jnp.float32),
        pltpu.VMEM_SHARED((_NP,), jnp.float32),
        pltpu.SemaphoreType.DMA,
        pltpu.SemaphoreType.DMA,
        pltpu.SemaphoreType.DMA,
        pltpu.SemaphoreType.DMA,
    ],
    compiler_params=pltpu.CompilerParams(
        needs_layout_passes=False, use_tc_tiling_on_sc=False),
    name="gat_sc_step",
)(_sc_body)


def _prep_body(z_ref, wa1_ref, wa2_ref, ba_ref, t0_ref,
               s1m_ref, s2b_ref, mv_ref, ft_ref):
  z = z_ref[...]
  s1 = jnp.sum(z * wa1_ref[...], axis=1, keepdims=True)
  s2b = jnp.sum(z * wa2_ref[...], axis=1, keepdims=True) + ba_ref[0, 0]
  m = jnp.maximum(jnp.max(s1) + jnp.max(s2b), 0.0)
  s1m_ref[...] = s1 - m
  s2b_ref[...] = s2b
  mv_ref[...] = jnp.full((1, 16), m, jnp.float32)
  ft_ref[...] = z * t0_ref[0, 0]


_tc_prep = pl.pallas_call(
    _prep_body,
    out_shape=[
        jax.ShapeDtypeStruct((_NP, 1), jnp.float32),
        jax.ShapeDtypeStruct((_NP, 1), jnp.float32),
        jax.ShapeDtypeStruct((1, 16), jnp.float32),
        jax.ShapeDtypeStruct((_NP, _D), jnp.float32),
    ],
    in_specs=[
        pl.BlockSpec(memory_space=pltpu.VMEM),
        pl.BlockSpec(memory_space=pltpu.VMEM),
        pl.BlockSpec(memory_space=pltpu.VMEM),
        pl.BlockSpec(memory_space=pltpu.SMEM),
        pl.BlockSpec(memory_space=pltpu.SMEM),
    ],
    name="gat_tc_prep",
)


def _comb_body(fp_ref, dp_ref, ft_ref, wa1_ref, wa2_ref, ba_ref, tk_ref,
               z3_ref, fto_ref, s1m_ref, s2b_ref, mv_ref):
  fsum = jnp.concatenate([fp_ref[0], fp_ref[1]], axis=1)
  den = dp_ref[0]
  feats = jnp.where(den > 0, fsum / den, 0.0)
  z3_ref[0] = feats[:, :_DH]
  z3_ref[1] = feats[:, _DH:]
  fto_ref[...] = ft_ref[...] + tk_ref[0, 0] * feats
  s1 = jnp.sum(feats * wa1_ref[...], axis=1, keepdims=True)
  s2b = jnp.sum(feats * wa2_ref[...], axis=1, keepdims=True) + ba_ref[0, 0]
  m = jnp.maximum(jnp.max(s1) + jnp.max(s2b), 0.0)
  s1m_ref[...] = s1 - m
  s2b_ref[...] = s2b
  mv_ref[...] = jnp.full((1, 16), m, jnp.float32)


_tc_comb = pl.pallas_call(
    _comb_body,
    out_shape=[
        jax.ShapeDtypeStruct((2, _NP, _DH), jnp.float32),
        jax.ShapeDtypeStruct((_NP, _D), jnp.float32),
        jax.ShapeDtypeStruct((_NP, 1), jnp.float32),
        jax.ShapeDtypeStruct((_NP, 1), jnp.float32),
        jax.ShapeDtypeStruct((1, 16), jnp.float32),
    ],
    in_specs=[
        pl.BlockSpec(memory_space=pltpu.VMEM),
        pl.BlockSpec(memory_space=pltpu.VMEM),
        pl.BlockSpec(memory_space=pltpu.VMEM),
        pl.BlockSpec(memory_space=pltpu.VMEM),
        pl.BlockSpec(memory_space=pltpu.VMEM),
        pl.BlockSpec(memory_space=pltpu.SMEM),
        pl.BlockSpec(memory_space=pltpu.SMEM),
    ],
    compiler_params=pltpu.CompilerParams(vmem_limit_bytes=100 * 1024 * 1024),
    name="gat_tc_comb",
)


def _last_body(fp_ref, dp_ref, ft_ref, tk_ref, fto_ref):
  fsum = jnp.concatenate([fp_ref[0], fp_ref[1]], axis=1)
  den = dp_ref[0]
  feats = jnp.where(den > 0, fsum / den, 0.0)
  fto_ref[...] = ft_ref[...] + tk_ref[0, 0] * feats


_tc_last = pl.pallas_call(
    _last_body,
    out_shape=jax.ShapeDtypeStruct((_NP, _D), jnp.float32),
    in_specs=[
        pl.BlockSpec(memory_space=pltpu.VMEM),
        pl.BlockSpec(memory_space=pltpu.VMEM),
        pl.BlockSpec(memory_space=pltpu.VMEM),
        pl.BlockSpec(memory_space=pltpu.SMEM),
    ],
    name="gat_tc_last",
)


@jax.jit
def _gat(features, edge_index, Wa, ba, temp):
  z0 = jnp.zeros((_NP, _D), jnp.float32).at[:_N].set(features)
  pad = jnp.full((_EP - _E,), _N, jnp.int32)
  src2 = jnp.concatenate([edge_index[0], pad]).reshape(_EP // _CH, _CH)
  dst2 = jnp.concatenate([edge_index[1], pad]).reshape(_EP // _CH, _CH)
  wa1 = Wa[:, :_D]
  wa2 = Wa[:, _D:]
  ba2 = ba.reshape(1, 1)
  s1m, s2b, mv, ft = _tc_prep(z0, wa1, wa2, ba2, temp[0].reshape(1, 1))
  z3 = jnp.stack([z0[:, :_DH], z0[:, _DH:]])
  for k in range(_K):
    fp, dp = _sc_step(z3, s1m.reshape(-1), s2b.reshape(-1), mv.reshape(-1),
                      src2, dst2)
    tk = temp[k + 1].reshape(1, 1)
    if k < _K - 1:
      z3, ft, s1m, s2b, mv = _tc_comb(fp, dp.reshape(2, _NP, 1), ft,
                                      wa1, wa2, ba2, tk)
    else:
      ft = _tc_last(fp, dp.reshape(2, _NP, 1), ft, tk)
  return ft[:_N]


def kernel(features, edge_index, Wa, ba, temp):
  return _gat(features, edge_index, Wa, ba, temp)


# speedup vs baseline: 12.5863x; 1.0622x over previous
import functools

import jax
import jax.numpy as jnp
from jax import lax
from jax.experimental import pallas as pl
from jax.experimental.pallas import tpu as pltpu
from jax.experimental.pallas import tpu_sc as plsc

_N = 10000
_E = 320000
_D = 128
_DH = _D // 2
_K = 3
_NP = 10240
_EP = 327680
_EPT = _EP // 16
_CH = 128
_NCH = _EPT // _CH
_RPT = _NP // 16


def _sc_body(z_hbm, s1m_hbm, s2b_hbm, mv_hbm, src_hbm, dst_hbm,
             outf_hbm, outd_hbm,
             s1v, s2v, srcA, dstA, srcB, dstB, mvv,
             exch0, exch1, exch2, exch3, rows0, rows1, rows2, rows3,
             zb, facc, dacc,
             gsem0, gsem1, gsem2, gsem3, ssem0, ssem1, ssem2, ssem3,
             isemA, isemB):
  cid = lax.axis_index("c")
  tid = lax.axis_index("s")
  zero16 = jnp.zeros((16,), jnp.float32)

  def _zrow(r, c):
    for i in range(_DH // 16):
      rows0[r, pl.ds(i * 16, 16)] = zero16
    return c
  lax.fori_loop(0, _CH, _zrow, 0)

  def _zzb(j, c):
    zb[pl.ds(j * 16, 16)] = zero16
    return c
  lax.fori_loop(0, _RPT // 16, _zzb, 0)

  for b in range(_RPT // _CH):
    pltpu.sync_copy(rows0, facc.at[pl.ds(tid * _RPT + b * _CH, _CH)])
  pltpu.sync_copy(zb, dacc.at[pl.ds(tid * _RPT, _RPT)])

  pltpu.sync_copy(s1m_hbm, s1v)
  pltpu.sync_copy(s2b_hbm, s2v)
  pltpu.sync_copy(mv_hbm, mvv)
  plsc.subcore_barrier()

  mvec = mvv[...]
  nmv = -mvec
  cbase = tid * _NCH

  def _ex_into(srcq, dstq, s, exch):
    for i in range(_CH // 16):
      si = srcq[s, pl.ds(i * 16, 16)]
      di = dstq[s, pl.ds(i * 16, 16)]
      t = plsc.load_gather(s1v, [si]) + plsc.load_gather(s2v, [di])
      exch[pl.ds(i * 16, 16)] = jnp.exp(
          jnp.where(t > nmv, t, t * 0.01 - mvec * 0.99))

  def _scale(rows, exch):
    def _srow(g, cc):
      wv = exch[pl.ds(g * 16, 16)]
      for jj in range(16):
        r = g * 16 + jj
        w = wv[jj]
        for i in range(_DH // 16):
          rows[r, pl.ds(i * 16, 16)] = rows[r, pl.ds(i * 16, 16)] * w
      return cc
    lax.fori_loop(0, _CH // 16, _srow, 0)

  def _drain(rows, exch, ssem):
    pltpu.make_async_copy(rows, facc.at[pl.ds(0, _CH)], ssem).wait()
    pltpu.make_async_copy(exch, dacc.at[pl.ds(0, _CH)], ssem).wait()

  slots = list(zip((rows0, rows1, rows2, rows3),
                   (exch0, exch1, exch2, exch3),
                   (gsem0, gsem1, gsem2, gsem3),
                   (ssem0, ssem1, ssem2, ssem3)))
  nslots = len(slots)
  nbody = _NCH // (2 * nslots)

  def _fetch(srcq, dstq, q0, isem):
    pltpu.async_copy(src_hbm.at[pl.ds(cbase + q0, nslots)], srcq, isem)
    pltpu.async_copy(dst_hbm.at[pl.ds(cbase + q0, nslots)], dstq, isem)

  def _idrain(srcq, dstq, isem):
    pltpu.make_async_copy(src_hbm.at[pl.ds(cbase, nslots)], srcq, isem).wait()
    pltpu.make_async_copy(dst_hbm.at[pl.ds(cbase, nslots)], dstq, isem).wait()

  def _run_quad(srcq, dstq, q, first, last):
    gds = []
    for s, (rows, exch, gsem, ssem) in enumerate(slots):
      if not first:
        _drain(rows, exch, ssem)
      else:
        @pl.when(q > 0)
        def _():
          _drain(rows, exch, ssem)
      _ex_into(srcq, dstq, s, exch)
      gds.append(
          pltpu.async_copy(z_hbm.at[cid].at[srcq.at[s]], rows, gsem))
    for s, (rows, exch, gsem, ssem) in enumerate(slots):
      gds[s].wait()
      _scale(rows, exch)
      pltpu.async_copy(rows, facc.at[dstq.at[s]], ssem, add=True)
      pltpu.async_copy(exch, dacc.at[dstq.at[s]], ssem, add=True)

  _fetch(srcA, dstA, 0, isemA)

  def _body(h, carry):
    qa = h * 2 * nslots
    qb = qa + nslots
    _idrain(srcA, dstA, isemA)
    _fetch(srcB, dstB, qb, isemB)
    _run_quad(srcA, dstA, h, True, False)
    _idrain(srcB, dstB, isemB)
    @pl.when(h < nbody - 1)
    def _():
      _fetch(srcA, dstA, qb + nslots, isemA)
    _run_quad(srcB, dstB, h, False, False)
    return carry

  lax.fori_loop(0, nbody, _body, 0)
  for rows, exch, _, ssem in slots:
    _drain(rows, exch, ssem)
  plsc.subcore_barrier()

  for b in range(_RPT // _CH):
    off = tid * _RPT + b * _CH
    pltpu.sync_copy(facc.at[pl.ds(off, _CH)], rows0)
    pltpu.sync_copy(rows0, outf_hbm.at[cid, pl.ds(off, _CH)])
  pltpu.sync_copy(dacc.at[pl.ds(tid * _RPT, _RPT)], zb)
  pltpu.sync_copy(zb, outd_hbm.at[cid, pl.ds(tid * _RPT, _RPT)])


_sc_step = functools.partial(
    pl.kernel,
    out_type=[
        jax.ShapeDtypeStruct((2, _NP, _DH), jnp.float32),
        jax.ShapeDtypeStruct((2, _NP), jnp.float32),
    ],
    mesh=plsc.VectorSubcoreMesh(core_axis_name="c", subcore_axis_name="s"),
    scratch_types=[
        pltpu.VMEM((_NP,), jnp.float32),
        pltpu.VMEM((_NP,), jnp.float32),
        pltpu.VMEM((4, _CH), jnp.int32),
        pltpu.VMEM((4, _CH), jnp.int32),
        pltpu.VMEM((4, _CH), jnp.int32),
        pltpu.VMEM((4, _CH), jnp.int32),
        pltpu.VMEM((16,), jnp.float32),
        pltpu.VMEM((_CH,), jnp.float32),
        pltpu.VMEM((_CH,), jnp.float32),
        pltpu.VMEM((_CH,), jnp.float32),
        pltpu.VMEM((_CH,), jnp.float32),
        pltpu.VMEM((_CH, _DH), jnp.float32),
        pltpu.VMEM((_CH, _DH), jnp.float32),
        pltpu.VMEM((_CH, _DH), jnp.float32),
        pltpu.VMEM((_CH, _DH), jnp.float32),
        pltpu.VMEM((_RPT,), jnp.float32),
        pltpu.VMEM_SHARED((_NP, _DH), jnp.float32),
        pltpu.VMEM_SHARED((_NP,), jnp.float32),
        pltpu.SemaphoreType.DMA,
        pltpu.SemaphoreType.DMA,
        pltpu.SemaphoreType.DMA,
        pltpu.SemaphoreType.DMA,
        pltpu.SemaphoreType.DMA,
        pltpu.SemaphoreType.DMA,
        pltpu.SemaphoreType.DMA,
        pltpu.SemaphoreType.DMA,
        pltpu.SemaphoreType.DMA,
        pltpu.SemaphoreType.DMA,
    ],
    compiler_params=pltpu.CompilerParams(
        needs_layout_passes=False, use_tc_tiling_on_sc=False),
    name="gat_sc_step",
)(_sc_body)


def _prep_body(z_ref, wa1_ref, wa2_ref, ba_ref, t0_ref,
               s1m_ref, s2b_ref, mv_ref, ft_ref):
  z = z_ref[...]
  s1 = jnp.sum(z * wa1_ref[...], axis=1, keepdims=True)
  s2b = jnp.sum(z * wa2_ref[...], axis=1, keepdims=True) + ba_ref[0, 0]
  m = jnp.maximum(jnp.max(s1) + jnp.max(s2b), 0.0)
  s1m_ref[...] = s1 - m
  s2b_ref[...] = s2b
  mv_ref[...] = jnp.full((1, 16), m, jnp.float32)
  ft_ref[...] = z * t0_ref[0, 0]


_tc_prep = pl.pallas_call(
    _prep_body,
    out_shape=[
        jax.ShapeDtypeStruct((_NP, 1), jnp.float32),
        jax.ShapeDtypeStruct((_NP, 1), jnp.float32),
        jax.ShapeDtypeStruct((1, 16), jnp.float32),
        jax.ShapeDtypeStruct((_NP, _D), jnp.float32),
    ],
    in_specs=[
        pl.BlockSpec(memory_space=pltpu.VMEM),
        pl.BlockSpec(memory_space=pltpu.VMEM),
        pl.BlockSpec(memory_space=pltpu.VMEM),
        pl.BlockSpec(memory_space=pltpu.SMEM),
        pl.BlockSpec(memory_space=pltpu.SMEM),
    ],
    name="gat_tc_prep",
)


def _comb_body(fp_ref, dp_ref, ft_ref, wa1_ref, wa2_ref, ba_ref, tk_ref,
               z3_ref, fto_ref, s1m_ref, s2b_ref, mv_ref):
  fsum = jnp.concatenate([fp_ref[0], fp_ref[1]], axis=1)
  den = dp_ref[0]
  feats = jnp.where(den > 0, fsum / den, 0.0)
  z3_ref[0] = feats[:, :_DH]
  z3_ref[1] = feats[:, _DH:]
  fto_ref[...] = ft_ref[...] + tk_ref[0, 0] * feats
  s1 = jnp.sum(feats * wa1_ref[...], axis=1, keepdims=True)
  s2b = jnp.sum(feats * wa2_ref[...], axis=1, keepdims=True) + ba_ref[0, 0]
  m = jnp.maximum(jnp.max(s1) + jnp.max(s2b), 0.0)
  s1m_ref[...] = s1 - m
  s2b_ref[...] = s2b
  mv_ref[...] = jnp.full((1, 16), m, jnp.float32)


_tc_comb = pl.pallas_call(
    _comb_body,
    out_shape=[
        jax.ShapeDtypeStruct((2, _NP, _DH), jnp.float32),
        jax.ShapeDtypeStruct((_NP, _D), jnp.float32),
        jax.ShapeDtypeStruct((_NP, 1), jnp.float32),
        jax.ShapeDtypeStruct((_NP, 1), jnp.float32),
        jax.ShapeDtypeStruct((1, 16), jnp.float32),
    ],
    in_specs=[
        pl.BlockSpec(memory_space=pltpu.VMEM),
        pl.BlockSpec(memory_space=pltpu.VMEM),
        pl.BlockSpec(memory_space=pltpu.VMEM),
        pl.BlockSpec(memory_space=pltpu.VMEM),
        pl.BlockSpec(memory_space=pltpu.VMEM),
        pl.BlockSpec(memory_space=pltpu.SMEM),
        pl.BlockSpec(memory_space=pltpu.SMEM),
    ],
    compiler_params=pltpu.CompilerParams(vmem_limit_bytes=100 * 1024 * 1024),
    name="gat_tc_comb",
)


def _last_body(fp_ref, dp_ref, ft_ref, tk_ref, fto_ref):
  fsum = jnp.concatenate([fp_ref[0], fp_ref[1]], axis=1)
  den = dp_ref[0]
  feats = jnp.where(den > 0, fsum / den, 0.0)
  fto_ref[...] = ft_ref[...] + tk_ref[0, 0] * feats


_tc_last = pl.pallas_call(
    _last_body,
    out_shape=jax.ShapeDtypeStruct((_NP, _D), jnp.float32),
    in_specs=[
        pl.BlockSpec(memory_space=pltpu.VMEM),
        pl.BlockSpec(memory_space=pltpu.VMEM),
        pl.BlockSpec(memory_space=pltpu.VMEM),
        pl.BlockSpec(memory_space=pltpu.SMEM),
    ],
    name="gat_tc_last",
)


@jax.jit
def _gat(features, edge_index, Wa, ba, temp):
  z0 = jnp.zeros((_NP, _D), jnp.float32).at[:_N].set(features)
  pad = jnp.full((_EP - _E,), _N, jnp.int32)
  src2 = jnp.concatenate([edge_index[0], pad]).reshape(_EP // _CH, _CH)
  dst2 = jnp.concatenate([edge_index[1], pad]).reshape(_EP // _CH, _CH)
  wa1 = Wa[:, :_D]
  wa2 = Wa[:, _D:]
  ba2 = ba.reshape(1, 1)
  s1m, s2b, mv, ft = _tc_prep(z0, wa1, wa2, ba2, temp[0].reshape(1, 1))
  z3 = jnp.stack([z0[:, :_DH], z0[:, _DH:]])
  for k in range(_K):
    fp, dp = _sc_step(z3, s1m.reshape(-1), s2b.reshape(-1), mv.reshape(-1),
                      src2, dst2)
    tk = temp[k + 1].reshape(1, 1)
    if k < _K - 1:
      z3, ft, s1m, s2b, mv = _tc_comb(fp, dp.reshape(2, _NP, 1), ft,
                                      wa1, wa2, ba2, tk)
    else:
      ft = _tc_last(fp, dp.reshape(2, _NP, 1), ft, tk)
  return ft[:_N]


def kernel(features, edge_index, Wa, ba, temp):
  return _gat(features, edge_index, Wa, ba, temp)


# speedup vs baseline: 13.3236x; 1.0586x over previous
import functools

import jax
import jax.numpy as jnp
from jax import lax
from jax.experimental import pallas as pl
from jax.experimental.pallas import tpu as pltpu
from jax.experimental.pallas import tpu_sc as plsc

_N = 10000
_E = 320000
_D = 128
_DH = _D // 2
_K = 3
_NP = 10240
_EP = 327680
_EPT = _EP // 16
_CH = 128
_NCH = _EPT // _CH
_RPT = _NP // 16


def _sc_body(z_hbm, s1m_hbm, s2b_hbm, mv_hbm, src_hbm, dst_hbm,
             outf_hbm, outd_hbm,
             s1v, s2v, srcA, dstA, srcB, dstB, mvv,
             exch0, exch1, exch2, exch3, rows0, rows1, rows2, rows3,
             zb, facc, dacc,
             gsem0, gsem1, gsem2, gsem3, ssem0, ssem1, ssem2, ssem3,
             isemA, isemB):
  cid = lax.axis_index("c")
  tid = lax.axis_index("s")
  zero16 = jnp.zeros((16,), jnp.float32)

  def _zrow(r, c):
    for i in range(_DH // 16):
      rows0[r, pl.ds(i * 16, 16)] = zero16
    return c
  lax.fori_loop(0, _CH, _zrow, 0)

  def _zzb(j, c):
    zb[pl.ds(j * 16, 16)] = zero16
    return c
  lax.fori_loop(0, _RPT // 16, _zzb, 0)

  for b in range(_RPT // _CH):
    pltpu.sync_copy(rows0, facc.at[pl.ds(tid * _RPT + b * _CH, _CH)])
  pltpu.sync_copy(zb, dacc.at[pl.ds(tid * _RPT, _RPT)])

  pltpu.sync_copy(s1m_hbm, s1v)
  pltpu.sync_copy(s2b_hbm, s2v)
  pltpu.sync_copy(mv_hbm, mvv)
  plsc.subcore_barrier()

  mvec = mvv[...]
  nmv = -mvec
  cbase = tid * _NCH

  def _ex_into(srcq, dstq, s, exch):
    for i in range(_CH // 16):
      si = srcq[s, pl.ds(i * 16, 16)]
      di = dstq[s, pl.ds(i * 16, 16)]
      t = plsc.load_gather(s1v, [si]) + plsc.load_gather(s2v, [di])
      exch[pl.ds(i * 16, 16)] = jnp.exp(
          jnp.where(t > nmv, t, t * 0.01 - mvec * 0.99))

  def _scale(rows, exch):
    def _srow(g, cc):
      wv = exch[pl.ds(g * 16, 16)]
      for jj in range(16):
        r = g * 16 + jj
        w = wv[jj]
        for i in range(_DH // 16):
          rows[r, pl.ds(i * 16, 16)] = rows[r, pl.ds(i * 16, 16)] * w
      return cc
    lax.fori_loop(0, _CH // 16, _srow, 0, unroll=2)

  def _drain(rows, exch, ssem):
    pltpu.make_async_copy(rows, facc.at[pl.ds(0, _CH)], ssem).wait()
    @pl.when(cid == 0)
    def _():
      pltpu.make_async_copy(exch, dacc.at[pl.ds(0, _CH)], ssem).wait()

  slots = list(zip((rows0, rows1, rows2, rows3),
                   (exch0, exch1, exch2, exch3),
                   (gsem0, gsem1, gsem2, gsem3),
                   (ssem0, ssem1, ssem2, ssem3)))
  nslots = len(slots)
  nbody = _NCH // (2 * nslots)

  def _fetch(srcq, dstq, q0, isem):
    pltpu.async_copy(src_hbm.at[pl.ds(cbase + q0, nslots)], srcq, isem)
    pltpu.async_copy(dst_hbm.at[pl.ds(cbase + q0, nslots)], dstq, isem)

  def _idrain(srcq, dstq, isem):
    pltpu.make_async_copy(src_hbm.at[pl.ds(cbase, nslots)], srcq, isem).wait()
    pltpu.make_async_copy(dst_hbm.at[pl.ds(cbase, nslots)], dstq, isem).wait()

  def _run_quad(srcq, dstq, q, first, last):
    gds = []
    for s, (rows, exch, gsem, ssem) in enumerate(slots):
      if not first:
        _drain(rows, exch, ssem)
      else:
        @pl.when(q > 0)
        def _():
          _drain(rows, exch, ssem)
      _ex_into(srcq, dstq, s, exch)
      gds.append(
          pltpu.async_copy(z_hbm.at[cid].at[srcq.at[s]], rows, gsem))
    for s, (rows, exch, gsem, ssem) in enumerate(slots):
      gds[s].wait()
      _scale(rows, exch)
      pltpu.async_copy(rows, facc.at[dstq.at[s]], ssem, add=True)
      @pl.when(cid == 0)
      def _():
        pltpu.async_copy(exch, dacc.at[dstq.at[s]], ssem, add=True)

  _fetch(srcA, dstA, 0, isemA)

  def _body(h, carry):
    qa = h * 2 * nslots
    qb = qa + nslots
    _idrain(srcA, dstA, isemA)
    _fetch(srcB, dstB, qb, isemB)
    _run_quad(srcA, dstA, h, True, False)
    _idrain(srcB, dstB, isemB)
    @pl.when(h < nbody - 1)
    def _():
      _fetch(srcA, dstA, qb + nslots, isemA)
    _run_quad(srcB, dstB, h, False, False)
    return carry

  lax.fori_loop(0, nbody, _body, 0)
  for rows, exch, _, ssem in slots:
    _drain(rows, exch, ssem)
  plsc.subcore_barrier()

  for b in range(_RPT // _CH):
    off = tid * _RPT + b * _CH
    pltpu.sync_copy(facc.at[pl.ds(off, _CH)], rows0)
    pltpu.sync_copy(rows0, outf_hbm.at[cid, pl.ds(off, _CH)])
  pltpu.sync_copy(dacc.at[pl.ds(tid * _RPT, _RPT)], zb)
  pltpu.sync_copy(zb, outd_hbm.at[cid, pl.ds(tid * _RPT, _RPT)])


_sc_step = functools.partial(
    pl.kernel,
    out_type=[
        jax.ShapeDtypeStruct((2, _NP, _DH), jnp.float32),
        jax.ShapeDtypeStruct((2, _NP), jnp.float32),
    ],
    mesh=plsc.VectorSubcoreMesh(core_axis_name="c", subcore_axis_name="s"),
    scratch_types=[
        pltpu.VMEM((_NP,), jnp.float32),
        pltpu.VMEM((_NP,), jnp.float32),
        pltpu.VMEM((4, _CH), jnp.int32),
        pltpu.VMEM((4, _CH), jnp.int32),
        pltpu.VMEM((4, _CH), jnp.int32),
        pltpu.VMEM((4, _CH), jnp.int32),
        pltpu.VMEM((16,), jnp.float32),
        pltpu.VMEM((_CH,), jnp.float32),
        pltpu.VMEM((_CH,), jnp.float32),
        pltpu.VMEM((_CH,), jnp.float32),
        pltpu.VMEM((_CH,), jnp.float32),
        pltpu.VMEM((_CH, _DH), jnp.float32),
        pltpu.VMEM((_CH, _DH), jnp.float32),
        pltpu.VMEM((_CH, _DH), jnp.float32),
        pltpu.VMEM((_CH, _DH), jnp.float32),
        pltpu.VMEM((_RPT,), jnp.float32),
        pltpu.VMEM_SHARED((_NP, _DH), jnp.float32),
        pltpu.VMEM_SHARED((_NP,), jnp.float32),
        pltpu.SemaphoreType.DMA,
        pltpu.SemaphoreType.DMA,
        pltpu.SemaphoreType.DMA,
        pltpu.SemaphoreType.DMA,
        pltpu.SemaphoreType.DMA,
        pltpu.SemaphoreType.DMA,
        pltpu.SemaphoreType.DMA,
        pltpu.SemaphoreType.DMA,
        pltpu.SemaphoreType.DMA,
        pltpu.SemaphoreType.DMA,
    ],
    compiler_params=pltpu.CompilerParams(
        needs_layout_passes=False, use_tc_tiling_on_sc=False),
    name="gat_sc_step",
)(_sc_body)


def _prep_body(z_ref, wa1_ref, wa2_ref, ba_ref, t0_ref,
               s1m_ref, s2b_ref, mv_ref, ft_ref):
  z = z_ref[...]
  s1 = jnp.sum(z * wa1_ref[...], axis=1, keepdims=True)
  s2b = jnp.sum(z * wa2_ref[...], axis=1, keepdims=True) + ba_ref[0, 0]
  m = jnp.maximum(jnp.max(s1) + jnp.max(s2b), 0.0)
  s1m_ref[...] = s1 - m
  s2b_ref[...] = s2b
  mv_ref[...] = jnp.full((1, 16), m, jnp.float32)
  ft_ref[...] = z * t0_ref[0, 0]


_tc_prep = pl.pallas_call(
    _prep_body,
    out_shape=[
        jax.ShapeDtypeStruct((_NP, 1), jnp.float32),
        jax.ShapeDtypeStruct((_NP, 1), jnp.float32),
        jax.ShapeDtypeStruct((1, 16), jnp.float32),
        jax.ShapeDtypeStruct((_NP, _D), jnp.float32),
    ],
    in_specs=[
        pl.BlockSpec(memory_space=pltpu.VMEM),
        pl.BlockSpec(memory_space=pltpu.VMEM),
        pl.BlockSpec(memory_space=pltpu.VMEM),
        pl.BlockSpec(memory_space=pltpu.SMEM),
        pl.BlockSpec(memory_space=pltpu.SMEM),
    ],
    name="gat_tc_prep",
)


def _comb_body(fp_ref, dp_ref, ft_ref, wa1_ref, wa2_ref, ba_ref, tk_ref,
               z3_ref, fto_ref, s1m_ref, s2b_ref, mv_ref):
  fsum = jnp.concatenate([fp_ref[0], fp_ref[1]], axis=1)
  den = dp_ref[0]
  feats = jnp.where(den > 0, fsum / den, 0.0)
  z3_ref[0] = feats[:, :_DH]
  z3_ref[1] = feats[:, _DH:]
  fto_ref[...] = ft_ref[...] + tk_ref[0, 0] * feats
  s1 = jnp.sum(feats * wa1_ref[...], axis=1, keepdims=True)
  s2b = jnp.sum(feats * wa2_ref[...], axis=1, keepdims=True) + ba_ref[0, 0]
  m = jnp.maximum(jnp.max(s1) + jnp.max(s2b), 0.0)
  s1m_ref[...] = s1 - m
  s2b_ref[...] = s2b
  mv_ref[...] = jnp.full((1, 16), m, jnp.float32)


_tc_comb = pl.pallas_call(
    _comb_body,
    out_shape=[
        jax.ShapeDtypeStruct((2, _NP, _DH), jnp.float32),
        jax.ShapeDtypeStruct((_NP, _D), jnp.float32),
        jax.ShapeDtypeStruct((_NP, 1), jnp.float32),
        jax.ShapeDtypeStruct((_NP, 1), jnp.float32),
        jax.ShapeDtypeStruct((1, 16), jnp.float32),
    ],
    in_specs=[
        pl.BlockSpec(memory_space=pltpu.VMEM),
        pl.BlockSpec(memory_space=pltpu.VMEM),
        pl.BlockSpec(memory_space=pltpu.VMEM),
        pl.BlockSpec(memory_space=pltpu.VMEM),
        pl.BlockSpec(memory_space=pltpu.VMEM),
        pl.BlockSpec(memory_space=pltpu.SMEM),
        pl.BlockSpec(memory_space=pltpu.SMEM),
    ],
    compiler_params=pltpu.CompilerParams(vmem_limit_bytes=100 * 1024 * 1024),
    name="gat_tc_comb",
)


def _last_body(fp_ref, dp_ref, ft_ref, tk_ref, fto_ref):
  fsum = jnp.concatenate([fp_ref[0], fp_ref[1]], axis=1)
  den = dp_ref[0]
  feats = jnp.where(den > 0, fsum / den, 0.0)
  fto_ref[...] = ft_ref[...] + tk_ref[0, 0] * feats


_tc_last = pl.pallas_call(
    _last_body,
    out_shape=jax.ShapeDtypeStruct((_NP, _D), jnp.float32),
    in_specs=[
        pl.BlockSpec(memory_space=pltpu.VMEM),
        pl.BlockSpec(memory_space=pltpu.VMEM),
        pl.BlockSpec(memory_space=pltpu.VMEM),
        pl.BlockSpec(memory_space=pltpu.SMEM),
    ],
    name="gat_tc_last",
)


@jax.jit
def _gat(features, edge_index, Wa, ba, temp):
  z0 = jnp.zeros((_NP, _D), jnp.float32).at[:_N].set(features)
  pad = jnp.full((_EP - _E,), _N, jnp.int32)
  src2 = jnp.concatenate([edge_index[0], pad]).reshape(_EP // _CH, _CH)
  dst2 = jnp.concatenate([edge_index[1], pad]).reshape(_EP // _CH, _CH)
  wa1 = Wa[:, :_D]
  wa2 = Wa[:, _D:]
  ba2 = ba.reshape(1, 1)
  s1m, s2b, mv, ft = _tc_prep(z0, wa1, wa2, ba2, temp[0].reshape(1, 1))
  z3 = jnp.stack([z0[:, :_DH], z0[:, _DH:]])
  for k in range(_K):
    fp, dp = _sc_step(z3, s1m.reshape(-1), s2b.reshape(-1), mv.reshape(-1),
                      src2, dst2)
    tk = temp[k + 1].reshape(1, 1)
    if k < _K - 1:
      z3, ft, s1m, s2b, mv = _tc_comb(fp, dp.reshape(2, _NP, 1), ft,
                                      wa1, wa2, ba2, tk)
    else:
      ft = _tc_last(fp, dp.reshape(2, _NP, 1), ft, tk)
  return ft[:_N]


def kernel(features, edge_index, Wa, ba, temp):
  return _gat(features, edge_index, Wa, ba, temp)
